# baseline (speedup 1.0000x reference)
import functools

import jax
import jax.numpy as jnp
from jax import lax
from jax.experimental import pallas as pl
from jax.experimental.pallas import tpu as pltpu
from jax.experimental.pallas import tpu_sc as plsc

NC = 2
NS = 16
NW = NC * NS
LANES = 16


def _sc_avg_embed(x, table, b, l):
  d = table.shape[1]
  n_dreg = d // LANES
  bpt = b // NW
  assert b % NW == 0 and bpt <= 128 and bpt % LANES == 0
  assert l % 2 == 0 and d % LANES == 0
  half = (l + 1) // 2
  assert half <= 128
  scale = 1.0 / b
  mesh = plsc.VectorSubcoreMesh(core_axis_name="c", subcore_axis_name="s")

  @functools.partial(
      pl.kernel,
      mesh=mesh,
      out_type=jax.ShapeDtypeStruct((NC, l, d), jnp.float32),
      scratch_types=[
          pltpu.VMEM((bpt * l,), jnp.int32),
          pltpu.VMEM((l, bpt), jnp.int32),
          pltpu.VMEM((2 * bpt, d), jnp.float32),
          pltpu.VMEM((l, d), jnp.float32),
          pltpu.VMEM((2, half), jnp.int32),
          pltpu.VMEM_SHARED((l, d), jnp.float32),
          pltpu.SemaphoreType.DMA,
          pltpu.SemaphoreType.DMA,
      ],
      compiler_params=pltpu.CompilerParams(use_tc_tiling_on_sc=False,
                                           needs_layout_passes=False),
  )
  def body(x_hbm, table_hbm, out_hbm, xblk_v, xt_v, rows_v, stage_v, iref_v,
           shacc, sem0, sem1):
    cid = lax.axis_index("c")
    sid = lax.axis_index("s")
    wid = sid * NC + cid
    iota = lax.iota(jnp.int32, LANES)
    sems = (sem0, sem1)

    pltpu.sync_copy(x_hbm.at[pl.ds(wid * bpt * l, bpt * l)], xblk_v)

    starts = sorted({min(16 * k, half - LANES) for k in range(half // LANES + 1)})
    for r in range(2):
      for st in starts:
        iref_v[r, pl.ds(st, LANES)] = iota + (half * r + st)

    iota_l = iota * l
    def tr_body(j, _):
      for k in range(bpt // LANES):
        xt_v[j, pl.ds(k * LANES, LANES)] = plsc.load_gather(
            xblk_v, [iota_l + (k * LANES * l + j)])
      return 0

    lax.fori_loop(0, l, tr_body, 0)

    def fire(j, parity):
      pltpu.async_copy(
          table_hbm.at[xt_v.at[j]],
          rows_v.at[pl.ds(parity * bpt, bpt)],
          sems[parity],
      )

    def do_col(j, parity):
      base = parity * bpt

      @pl.when(j + 1 < l)
      def _fire_next():
        fire(j + 1, 1 - parity)

      pltpu.make_async_copy(
          table_hbm.at[pl.ds(0, bpt)],
          rows_v.at[pl.ds(base, bpt)],
          sems[parity],
      ).wait()

      def row_body(i, carry):
        return tuple(
            carry[r] + rows_v[base + i, pl.ds(r * LANES, LANES)]
            for r in range(n_dreg)
        )

      zeros = tuple(jnp.zeros((LANES,), jnp.float32) for _ in range(n_dreg))
      acc = lax.fori_loop(0, bpt, row_body, zeros, unroll=8)
      for r in range(n_dreg):
        stage_v[j, pl.ds(r * LANES, LANES)] = acc[r] * scale

    fire(0, 0)

    def pair_body(rp, _):
      do_col(2 * rp, 0)
      do_col(2 * rp + 1, 1)
      return 0

    lax.fori_loop(0, l // 2, pair_body, 0)

    @pl.when(sid == 0)
    def _init():
      pltpu.sync_copy(stage_v, shacc)

    plsc.subcore_barrier()

    @pl.when(sid != 0)
    def _reduce():
      pltpu.sync_copy(stage_v.at[pl.ds(0, half)], shacc.at[iref_v.at[0]],
                      add=True)
      pltpu.sync_copy(stage_v.at[pl.ds(half, l - half)],
                      shacc.at[iref_v.at[1]], add=True)

    plsc.subcore_barrier()

    @pl.when(sid == 0)
    def _writeout():
      pltpu.sync_copy(shacc, out_hbm.at[cid])

  return body(x, table)


def kernel(x, table):
  b, l = x.shape
  partials = _sc_avg_embed(x.astype(jnp.int32).reshape(-1), table, b, l)
  return partials[0] + partials[1]
